# baseline (device time: 103925 ns/iter reference)
import jax
import jax.numpy as jnp
from jax import lax
from jax.experimental import pallas as pl
from jax.experimental.pallas import tpu as pltpu

M = 8192
N = 2048
NOUT = 1024
STRIP = 1024
R = 256
JC = STRIP // R
NX = 3 * JC
NY = 2 * JC + 2
NZ = 2 * JC + 2
F32 = jnp.float32
BF16 = jnp.bfloat16


def kernel(x):
    def body(x_ref, out_ref,
             recv_x, recv_y, recv_z, cvt_in, send_buf,
             loc_x, loc_y, loc_z, out_sx, out_sy, out_sz,
             x_send_sems, x_recv_sems, y_send_sems, y_recv_sems,
             z_send_sems, z_recv_sems, cvt_sems,
             locx_sems, locy_sems, locz_sems,
             outx_sems, outy_sems, outz_sems):
        xi = lax.axis_index("x")
        yi = lax.axis_index("y")
        zi = lax.axis_index("z")
        peer_x = (1 - xi, yi, zi)
        peer_y = (xi, 1 - yi, zi)
        peer_z = (xi, yi, 1 - zi)

        my_cols = xi * NOUT
        peer_cols = (1 - xi) * NOUT

        eq = yi == zi
        s_myP = 2 * yi + zi
        s_Bz = jnp.where(eq, 4, 6)
        s_By = jnp.where(eq, 5, 7)
        s_yP = 2 * (1 - yi) + zi
        s_yB = jnp.where((1 - yi) == zi, 5, 7)
        s_zP = 2 * yi + (1 - zi)
        s_zB = jnp.where(yi == (1 - zi), 4, 6)
        s_zD = 2 * (1 - yi) + (1 - zi)

        x_strips = [s_myP, s_Bz, s_By]

        def rows_of(strip, j):
            return pl.ds(strip * STRIP + j * R, R)

        def x_rows(i):
            return rows_of(x_strips[i % 3], i // 3)

        def y_rows(m):
            if m < 8:
                return rows_of([s_yP, s_yB][m % 2], m // 2)
            return rows_of(s_zD, m - 6)

        def z_rows(n):
            if n < 8:
                return rows_of([s_zP, s_zB][n % 2], n // 2)
            return rows_of(s_zD, n - 8)

        barrier_sem = pltpu.get_barrier_semaphore()
        for peer in (peer_x, peer_y, peer_z):
            pl.semaphore_signal(
                barrier_sem, inc=1,
                device_id=peer, device_id_type=pl.DeviceIdType.MESH,
            )
        pl.semaphore_wait(barrier_sem, 3)

        streams = {
            "x": (loc_x, locx_sems, out_sx, outx_sems, NX, x_rows),
            "y": (loc_y, locy_sems, out_sy, outy_sems, NY, y_rows),
            "z": (loc_z, locz_sems, out_sz, outz_sems, NZ, z_rows),
        }
        stores = {"x": [None] * NX, "y": [None] * NY, "z": [None] * NZ}
        proc_pos = {"x": 0, "y": 0, "z": 0}

        def loc_load(stream, p):
            buf, lsems, _, _, _, rows_fn = streams[stream]
            q = proc_order[stream][p]
            pltpu.make_async_copy(
                x_ref.at[0, rows_fn(q), pl.ds(my_cols, NOUT)],
                buf.at[p % 2], lsems.at[p % 2]).start()

        def do_add(stream, q, recv_buf, slot):
            buf, lsems, obuf, osems, n_chunks, rows_fn = streams[stream]
            p = proc_pos[stream]
            proc_pos[stream] = p + 1
            assert proc_order[stream][p] == q
            pltpu.make_async_copy(
                x_ref.at[0, rows_fn(q), pl.ds(my_cols, NOUT)],
                buf.at[p % 2], lsems.at[p % 2]).wait()
            if p + 1 < n_chunks:
                loc_load(stream, p + 1)
            if p >= 2:
                stores[stream][p - 2].wait()
            obuf[p % 2] = (buf[p % 2] + recv_buf[slot].astype(F32)).astype(BF16)
            st = pltpu.make_async_copy(
                obuf.at[p % 2], out_ref.at[rows_fn(q), :], osems.at[p % 2])
            st.start()
            stores[stream][p] = st

        x_rdmas = [None] * NX
        fwd_y = [None] * NY
        fwd_z = [None] * NZ

        def fwd(src_buf, src_slot, dst_buf, dst_slot, send_sems, recv_sems,
                peer):
            r = pltpu.make_async_remote_copy(
                src_ref=src_buf.at[src_slot],
                dst_ref=dst_buf.at[dst_slot],
                send_sem=send_sems.at[dst_slot],
                recv_sem=recv_sems.at[dst_slot],
                device_id=peer,
                device_id_type=pl.DeviceIdType.MESH,
            )
            r.start()
            return r

        def cvt_load(i):
            pltpu.make_async_copy(
                x_ref.at[0, x_rows(i), pl.ds(peer_cols, NOUT)],
                cvt_in.at[i % 2], cvt_sems.at[i % 2]).start()

        def x_send(i):
            if i + 1 < NX:
                cvt_load(i + 1)
            pltpu.make_async_copy(
                x_ref.at[0, x_rows(i), pl.ds(peer_cols, NOUT)],
                cvt_in.at[i % 2], cvt_sems.at[i % 2]).wait()
            if i >= 2:
                x_rdmas[i - 2].wait_send()
            send_buf[i % 2] = cvt_in[i % 2].astype(BF16)
            r = pltpu.make_async_remote_copy(
                src_ref=send_buf.at[i % 2],
                dst_ref=recv_x.at[i],
                send_sem=x_send_sems.at[i % 2],
                recv_sem=x_recv_sems.at[i],
                device_id=peer_x,
                device_id_type=pl.DeviceIdType.MESH,
            )
            r.start()
            x_rdmas[i] = r

        def process_x(i):
            x_rdmas[i].wait_recv()
            c, j = i % 3, i // 3
            if c == 0:
                fwd_y[2 * j] = fwd(recv_x, i, recv_y, 2 * j,
                                   y_send_sems, y_recv_sems, peer_y)
                fwd_z[2 * j] = fwd(recv_x, i, recv_z, 2 * j,
                                   z_send_sems, z_recv_sems, peer_z)
            elif c == 1:
                fwd_z[2 * j + 1] = fwd(recv_x, i, recv_z, 2 * j + 1,
                                       z_send_sems, z_recv_sems, peer_z)
            else:
                fwd_y[2 * j + 1] = fwd(recv_x, i, recv_y, 2 * j + 1,
                                       y_send_sems, y_recv_sems, peer_y)
            do_add("x", i, recv_x, i)

        def process_y(m):
            fwd_y[m].wait_recv()
            if m < 8 and m % 2 == 0 and m // 2 < 2:
                j = m // 2
                fwd_z[8 + j] = fwd(recv_y, m, recv_z, 8 + j,
                                   z_send_sems, z_recv_sems, peer_z)
            do_add("y", m, recv_y, m)

        def process_z(n):
            fwd_z[n].wait_recv()
            if n < 8 and n % 2 == 0 and n // 2 >= 2:
                j = n // 2
                fwd_y[6 + j] = fwd(recv_z, n, recv_y, 6 + j,
                                   y_send_sems, y_recv_sems, peer_y)
            do_add("z", n, recv_z, n)

        def y_tick(m):
            if m < 8:
                i_m = 3 * (m // 2) + (0 if m % 2 == 0 else 2)
                return i_m + 2
            j = m - 6
            return 3 * j + 3 + 2

        def z_tick(n):
            if n < 8:
                c, j = n % 2, n // 2
                if c == 0:
                    return 3 * j + 2 + (0 if j >= 2 else 2)
                return 3 * j + 3 + 2
            j = n - 8
            return 3 * j + 3 + 2

        proc_order = {
            "x": list(range(NX)),
            "y": sorted(range(NY), key=lambda m: (y_tick(m), m)),
            "z": sorted(range(NZ), key=lambda n: (z_tick(n), n)),
        }

        cvt_load(0)
        loc_load("x", 0)
        loc_load("y", 0)
        loc_load("z", 0)

        for t in range(NX + 4):
            if t < NX:
                x_send(t)
            if 1 <= t <= NX:
                process_x(t - 1)
            for m in range(NY):
                if y_tick(m) == t:
                    process_y(m)
            for n in range(NZ):
                if z_tick(n) == t:
                    process_z(n)

        x_rdmas[NX - 2].wait_send()
        x_rdmas[NX - 1].wait_send()
        for r in fwd_y:
            r.wait_send()
        for r in fwd_z:
            r.wait_send()
        for s in ("x", "y", "z"):
            stores[s][-2].wait()
            stores[s][-1].wait()

    return pl.pallas_call(
        body,
        out_shape=jax.ShapeDtypeStruct((M, NOUT), BF16),
        in_specs=[pl.BlockSpec(memory_space=pltpu.HBM)],
        out_specs=pl.BlockSpec(memory_space=pltpu.HBM),
        scratch_shapes=[
            pltpu.VMEM((NX, R, NOUT), BF16),
            pltpu.VMEM((NY, R, NOUT), BF16),
            pltpu.VMEM((NZ, R, NOUT), BF16),
            pltpu.VMEM((2, R, NOUT), F32),
            pltpu.VMEM((2, R, NOUT), BF16),
            pltpu.VMEM((2, R, NOUT), F32),
            pltpu.VMEM((2, R, NOUT), F32),
            pltpu.VMEM((2, R, NOUT), F32),
            pltpu.VMEM((2, R, NOUT), BF16),
            pltpu.VMEM((2, R, NOUT), BF16),
            pltpu.VMEM((2, R, NOUT), BF16),
            pltpu.SemaphoreType.DMA((2,)),
            pltpu.SemaphoreType.DMA((NX,)),
            pltpu.SemaphoreType.DMA((NY,)),
            pltpu.SemaphoreType.DMA((NY,)),
            pltpu.SemaphoreType.DMA((NZ,)),
            pltpu.SemaphoreType.DMA((NZ,)),
            pltpu.SemaphoreType.DMA((2,)),
            pltpu.SemaphoreType.DMA((2,)),
            pltpu.SemaphoreType.DMA((2,)),
            pltpu.SemaphoreType.DMA((2,)),
            pltpu.SemaphoreType.DMA((2,)),
            pltpu.SemaphoreType.DMA((2,)),
            pltpu.SemaphoreType.DMA((2,)),
        ],
        compiler_params=pltpu.CompilerParams(
            collective_id=0, vmem_limit_bytes=48 * 1024 * 1024),
    )(x)


# device time: 98978 ns/iter; 1.0500x vs baseline; 1.0500x over previous
import jax
import jax.numpy as jnp
from jax import lax
from jax.experimental import pallas as pl
from jax.experimental.pallas import tpu as pltpu

M = 8192
N = 2048
NOUT = 1024
STRIP = 1024
R = 256
JC = STRIP // R
NX = 3 * JC
NY = 2 * JC
NZ = 3 * JC
F32 = jnp.float32
BF16 = jnp.bfloat16


def kernel(x):
    def body(x_ref, out_ref,
             recv_x, recv_y, recv_z, cvt_in, send_buf,
             loc_x, loc_y, loc_z, out_sx, out_sy, out_sz,
             x_send_sems, x_recv_sems, y_send_sems, y_recv_sems,
             z_send_sems, z_recv_sems, cvt_sems,
             locx_sems, locy_sems, locz_sems,
             outx_sems, outy_sems, outz_sems):
        xi = lax.axis_index("x")
        yi = lax.axis_index("y")
        zi = lax.axis_index("z")
        peer_x = (1 - xi, yi, zi)
        peer_y = (xi, 1 - yi, zi)
        peer_z = (xi, yi, 1 - zi)

        my_cols = xi * NOUT
        peer_cols = (1 - xi) * NOUT

        eq = yi == zi
        s_myP = 2 * yi + zi
        s_Bz = jnp.where(eq, 4, 6)
        s_By = jnp.where(eq, 5, 7)
        s_yP = 2 * (1 - yi) + zi
        s_yB = jnp.where((1 - yi) == zi, 5, 7)
        s_zP = 2 * yi + (1 - zi)
        s_zB = jnp.where(yi == (1 - zi), 4, 6)
        s_zD = 2 * (1 - yi) + (1 - zi)

        x_strips = [s_myP, s_Bz, s_By]
        y_strips = [s_yP, s_yB]
        z_strips = [s_zP, s_zB, s_zD]

        def rows_of(strip, j):
            return pl.ds(strip * STRIP + j * R, R)

        def x_rows(i):
            return rows_of(x_strips[i % 3], i // 3)

        def y_rows(m):
            return rows_of(y_strips[m % 2], m // 2)

        def z_rows(n):
            return rows_of(z_strips[n % 3], n // 3)

        streams = {
            "x": (loc_x, locx_sems, out_sx, outx_sems, NX, x_rows),
            "y": (loc_y, locy_sems, out_sy, outy_sems, NY, y_rows),
            "z": (loc_z, locz_sems, out_sz, outz_sems, NZ, z_rows),
        }
        stores = {"x": [None] * NX, "y": [None] * NY, "z": [None] * NZ}

        def loc_load(stream, q):
            buf, lsems, _, _, _, rows_fn = streams[stream]
            pltpu.make_async_copy(
                x_ref.at[0, rows_fn(q), pl.ds(my_cols, NOUT)],
                buf.at[q % 2], lsems.at[q % 2]).start()

        def do_add(stream, q, recv_buf, slot):
            buf, lsems, obuf, osems, n_chunks, rows_fn = streams[stream]
            pltpu.make_async_copy(
                x_ref.at[0, rows_fn(q), pl.ds(my_cols, NOUT)],
                buf.at[q % 2], lsems.at[q % 2]).wait()
            if q + 1 < n_chunks:
                loc_load(stream, q + 1)
            if q >= 2:
                stores[stream][q - 2].wait()
            obuf[q % 2] = (buf[q % 2] + recv_buf[slot].astype(F32)).astype(BF16)
            st = pltpu.make_async_copy(
                obuf.at[q % 2], out_ref.at[rows_fn(q), :], osems.at[q % 2])
            st.start()
            stores[stream][q] = st

        x_rdmas = [None] * NX
        fwd_y = [None] * NY
        fwd_z = [None] * NZ

        def fwd(src_buf, src_slot, dst_buf, dst_slot, send_sems, recv_sems,
                peer):
            r = pltpu.make_async_remote_copy(
                src_ref=src_buf.at[src_slot],
                dst_ref=dst_buf.at[dst_slot],
                send_sem=send_sems.at[dst_slot],
                recv_sem=recv_sems.at[dst_slot],
                device_id=peer,
                device_id_type=pl.DeviceIdType.MESH,
            )
            r.start()
            return r

        def cvt_load(i):
            pltpu.make_async_copy(
                x_ref.at[0, x_rows(i), pl.ds(peer_cols, NOUT)],
                cvt_in.at[i % 2], cvt_sems.at[i % 2]).start()

        def x_send(i):
            if i + 1 < NX:
                cvt_load(i + 1)
            pltpu.make_async_copy(
                x_ref.at[0, x_rows(i), pl.ds(peer_cols, NOUT)],
                cvt_in.at[i % 2], cvt_sems.at[i % 2]).wait()
            if i >= 2:
                x_rdmas[i - 2].wait_send()
            send_buf[i % 2] = cvt_in[i % 2].astype(BF16)
            r = pltpu.make_async_remote_copy(
                src_ref=send_buf.at[i % 2],
                dst_ref=recv_x.at[i],
                send_sem=x_send_sems.at[i % 2],
                recv_sem=x_recv_sems.at[i],
                device_id=peer_x,
                device_id_type=pl.DeviceIdType.MESH,
            )
            r.start()
            x_rdmas[i] = r

        def process_x(i):
            x_rdmas[i].wait_recv()
            c, j = i % 3, i // 3
            if c == 0:
                fwd_y[2 * j] = fwd(recv_x, i, recv_y, 2 * j,
                                   y_send_sems, y_recv_sems, peer_y)
                fwd_z[3 * j] = fwd(recv_x, i, recv_z, 3 * j,
                                   z_send_sems, z_recv_sems, peer_z)
            elif c == 1:
                fwd_z[3 * j + 1] = fwd(recv_x, i, recv_z, 3 * j + 1,
                                       z_send_sems, z_recv_sems, peer_z)
            else:
                fwd_y[2 * j + 1] = fwd(recv_x, i, recv_y, 2 * j + 1,
                                       y_send_sems, y_recv_sems, peer_y)
            do_add("x", i, recv_x, i)

        def process_y(m):
            fwd_y[m].wait_recv()
            if m % 2 == 0:
                j = m // 2
                fwd_z[3 * j + 2] = fwd(recv_y, m, recv_z, 3 * j + 2,
                                       z_send_sems, z_recv_sems, peer_z)
            do_add("y", m, recv_y, m)

        def process_z(n):
            fwd_z[n].wait_recv()
            do_add("z", n, recv_z, n)

        def y_tick(m):
            i_m = 3 * (m // 2) + (0 if m % 2 == 0 else 2)
            return i_m + 2

        def z_tick(n):
            c, j = n % 3, n // 3
            return (3 * j + 2 if c == 0 else 3 * j + 3) + 2

        cvt_load(0)
        loc_load("x", 0)
        loc_load("y", 0)
        loc_load("z", 0)

        barrier_sem = pltpu.get_barrier_semaphore()
        for peer in (peer_x, peer_y, peer_z):
            pl.semaphore_signal(
                barrier_sem, inc=1,
                device_id=peer, device_id_type=pl.DeviceIdType.MESH,
            )
        pl.semaphore_wait(barrier_sem, 3)

        for t in range(NX + 4):
            if t < NX:
                x_send(t)
            if 1 <= t <= NX:
                process_x(t - 1)
            for m in range(NY):
                if y_tick(m) == t:
                    process_y(m)
            for n in range(NZ):
                if z_tick(n) == t:
                    process_z(n)

        x_rdmas[NX - 2].wait_send()
        x_rdmas[NX - 1].wait_send()
        for r in fwd_y:
            r.wait_send()
        for r in fwd_z:
            r.wait_send()
        for s in ("x", "y", "z"):
            stores[s][-2].wait()
            stores[s][-1].wait()

    return pl.pallas_call(
        body,
        out_shape=jax.ShapeDtypeStruct((M, NOUT), BF16),
        in_specs=[pl.BlockSpec(memory_space=pltpu.HBM)],
        out_specs=pl.BlockSpec(memory_space=pltpu.HBM),
        scratch_shapes=[
            pltpu.VMEM((NX, R, NOUT), BF16),
            pltpu.VMEM((NY, R, NOUT), BF16),
            pltpu.VMEM((NZ, R, NOUT), BF16),
            pltpu.VMEM((2, R, NOUT), F32),
            pltpu.VMEM((2, R, NOUT), BF16),
            pltpu.VMEM((2, R, NOUT), F32),
            pltpu.VMEM((2, R, NOUT), F32),
            pltpu.VMEM((2, R, NOUT), F32),
            pltpu.VMEM((2, R, NOUT), BF16),
            pltpu.VMEM((2, R, NOUT), BF16),
            pltpu.VMEM((2, R, NOUT), BF16),
            pltpu.SemaphoreType.DMA((2,)),
            pltpu.SemaphoreType.DMA((NX,)),
            pltpu.SemaphoreType.DMA((NY,)),
            pltpu.SemaphoreType.DMA((NY,)),
            pltpu.SemaphoreType.DMA((NZ,)),
            pltpu.SemaphoreType.DMA((NZ,)),
            pltpu.SemaphoreType.DMA((2,)),
            pltpu.SemaphoreType.DMA((2,)),
            pltpu.SemaphoreType.DMA((2,)),
            pltpu.SemaphoreType.DMA((2,)),
            pltpu.SemaphoreType.DMA((2,)),
            pltpu.SemaphoreType.DMA((2,)),
            pltpu.SemaphoreType.DMA((2,)),
        ],
        compiler_params=pltpu.CompilerParams(
            collective_id=0, vmem_limit_bytes=48 * 1024 * 1024),
    )(x)


# device time: 95758 ns/iter; 1.0853x vs baseline; 1.0336x over previous
import jax
import jax.numpy as jnp
from jax import lax
from jax.experimental import pallas as pl
from jax.experimental.pallas import tpu as pltpu

M = 8192
N = 2048
NOUT = 1024
STRIP = 1024
R = 256
JC = STRIP // R
NX = 3 * JC
NY = 2 * JC
NZ = 3 * JC
F32 = jnp.float32
BF16 = jnp.bfloat16


def kernel(x):
    def body(x_ref, out_ref,
             recv_x, recv_y, recv_z, cvt_in, send_buf,
             loc_x, loc_y, loc_z, out_sx, out_sy, out_sz,
             x_send_sems, x_recv_sems, y_send_sems, y_recv_sems,
             z_send_sems, z_recv_sems, cvt_sems,
             locx_sems, locy_sems, locz_sems,
             outx_sems, outy_sems, outz_sems):
        xi = lax.axis_index("x")
        yi = lax.axis_index("y")
        zi = lax.axis_index("z")
        peer_x = (1 - xi, yi, zi)
        peer_y = (xi, 1 - yi, zi)
        peer_z = (xi, yi, 1 - zi)

        my_cols = xi * NOUT
        peer_cols = (1 - xi) * NOUT

        eq = yi == zi
        s_myP = 2 * yi + zi
        s_Bz = jnp.where(eq, 4, 6)
        s_By = jnp.where(eq, 5, 7)
        s_yP = 2 * (1 - yi) + zi
        s_yB = jnp.where((1 - yi) == zi, 5, 7)
        s_zP = 2 * yi + (1 - zi)
        s_zB = jnp.where(yi == (1 - zi), 4, 6)
        s_zD = 2 * (1 - yi) + (1 - zi)

        x_strips = [s_myP, s_Bz, s_By]
        y_strips = [s_yP, s_yB]
        z_strips = [s_zP, s_zB, s_zD]

        def rows_of(strip, j):
            return pl.ds(strip * STRIP + j * R, R)

        def x_rows(i):
            return rows_of(x_strips[i % 3], i // 3)

        def y_rows(m):
            return rows_of(y_strips[m % 2], m // 2)

        def z_rows(n):
            return rows_of(z_strips[n % 3], n // 3)

        streams = {
            "x": (loc_x, locx_sems, out_sx, outx_sems, NX, x_rows),
            "y": (loc_y, locy_sems, out_sy, outy_sems, NY, y_rows),
            "z": (loc_z, locz_sems, out_sz, outz_sems, NZ, z_rows),
        }
        stores = {"x": [None] * NX, "y": [None] * NY, "z": [None] * NZ}
        proc_pos = {"x": 0, "y": 0, "z": 0}

        def loc_load(stream, p):
            buf, lsems, _, _, _, rows_fn = streams[stream]
            q = proc_order[stream][p]
            pltpu.make_async_copy(
                x_ref.at[0, rows_fn(q), pl.ds(my_cols, NOUT)],
                buf.at[p % 2], lsems.at[p % 2]).start()

        def do_add(stream, q, recv_buf, slot):
            buf, lsems, obuf, osems, n_chunks, rows_fn = streams[stream]
            p = proc_pos[stream]
            proc_pos[stream] = p + 1
            assert proc_order[stream][p] == q
            pltpu.make_async_copy(
                x_ref.at[0, rows_fn(q), pl.ds(my_cols, NOUT)],
                buf.at[p % 2], lsems.at[p % 2]).wait()
            if p + 1 < n_chunks:
                loc_load(stream, p + 1)
            if p >= 2:
                stores[stream][p - 2].wait()
            obuf[p % 2] = (buf[p % 2] + recv_buf[slot].astype(F32)).astype(BF16)
            st = pltpu.make_async_copy(
                obuf.at[p % 2], out_ref.at[rows_fn(q), :], osems.at[p % 2])
            st.start()
            stores[stream][p] = st

        x_rdmas = [None] * NX
        fwd_y = [None] * NY
        fwd_z = [None] * NZ

        def fwd(src_buf, src_slot, dst_buf, dst_slot, send_sems, recv_sems,
                peer):
            r = pltpu.make_async_remote_copy(
                src_ref=src_buf.at[src_slot],
                dst_ref=dst_buf.at[dst_slot],
                send_sem=send_sems.at[dst_slot],
                recv_sem=recv_sems.at[dst_slot],
                device_id=peer,
                device_id_type=pl.DeviceIdType.MESH,
            )
            r.start()
            return r

        def cvt_load(i):
            pltpu.make_async_copy(
                x_ref.at[0, x_rows(i), pl.ds(peer_cols, NOUT)],
                cvt_in.at[i % 2], cvt_sems.at[i % 2]).start()

        def x_send(i):
            if i + 1 < NX:
                cvt_load(i + 1)
            pltpu.make_async_copy(
                x_ref.at[0, x_rows(i), pl.ds(peer_cols, NOUT)],
                cvt_in.at[i % 2], cvt_sems.at[i % 2]).wait()
            if i >= 4:
                x_rdmas[i - 4].wait_send()
            send_buf[i % 4] = cvt_in[i % 2].astype(BF16)
            r = pltpu.make_async_remote_copy(
                src_ref=send_buf.at[i % 4],
                dst_ref=recv_x.at[i],
                send_sem=x_send_sems.at[i % 4],
                recv_sem=x_recv_sems.at[i],
                device_id=peer_x,
                device_id_type=pl.DeviceIdType.MESH,
            )
            r.start()
            x_rdmas[i] = r

        def process_x(i):
            x_rdmas[i].wait_recv()
            c, j = i % 3, i // 3
            if c == 0:
                fwd_y[2 * j] = fwd(recv_x, i, recv_y, 2 * j,
                                   y_send_sems, y_recv_sems, peer_y)
                fwd_z[3 * j] = fwd(recv_x, i, recv_z, 3 * j,
                                   z_send_sems, z_recv_sems, peer_z)
            elif c == 1:
                fwd_z[3 * j + 1] = fwd(recv_x, i, recv_z, 3 * j + 1,
                                       z_send_sems, z_recv_sems, peer_z)
            else:
                fwd_y[2 * j + 1] = fwd(recv_x, i, recv_y, 2 * j + 1,
                                       y_send_sems, y_recv_sems, peer_y)
            do_add("x", i, recv_x, i)

        def process_y(m):
            fwd_y[m].wait_recv()
            if m % 2 == 0:
                j = m // 2
                fwd_z[3 * j + 2] = fwd(recv_y, m, recv_z, 3 * j + 2,
                                       z_send_sems, z_recv_sems, peer_z)
            do_add("y", m, recv_y, m)

        def process_z(n):
            fwd_z[n].wait_recv()
            do_add("z", n, recv_z, n)

        def y_tick(m):
            i_m = 3 * (m // 2) + (0 if m % 2 == 0 else 2)
            return i_m + 2 + (0 if m % 2 == 0 else 2)

        def z_tick(n):
            c, j = n % 3, n // 3
            return (3 * j + 2 if c == 0 else 3 * j + 3) + 2

        proc_order = {
            "x": list(range(NX)),
            "y": sorted(range(NY), key=lambda m: (y_tick(m), m)),
            "z": sorted(range(NZ), key=lambda n: (z_tick(n), n)),
        }

        cvt_load(0)
        loc_load("x", 0)
        loc_load("y", 0)
        loc_load("z", 0)

        barrier_sem = pltpu.get_barrier_semaphore()
        for peer in (peer_x, peer_y, peer_z):
            pl.semaphore_signal(
                barrier_sem, inc=1,
                device_id=peer, device_id_type=pl.DeviceIdType.MESH,
            )
        pl.semaphore_wait(barrier_sem, 3)

        for t in range(NX + 4):
            if t < NX:
                x_send(t)
            if 1 <= t <= NX:
                process_x(t - 1)
            for m in range(NY):
                if y_tick(m) == t:
                    process_y(m)
            for n in range(NZ):
                if z_tick(n) == t:
                    process_z(n)

        for i in range(NX - 4, NX):
            x_rdmas[i].wait_send()
        for r in fwd_y:
            r.wait_send()
        for r in fwd_z:
            r.wait_send()
        for s in ("x", "y", "z"):
            stores[s][-2].wait()
            stores[s][-1].wait()

    return pl.pallas_call(
        body,
        out_shape=jax.ShapeDtypeStruct((M, NOUT), BF16),
        in_specs=[pl.BlockSpec(memory_space=pltpu.HBM)],
        out_specs=pl.BlockSpec(memory_space=pltpu.HBM),
        scratch_shapes=[
            pltpu.VMEM((NX, R, NOUT), BF16),
            pltpu.VMEM((NY, R, NOUT), BF16),
            pltpu.VMEM((NZ, R, NOUT), BF16),
            pltpu.VMEM((2, R, NOUT), F32),
            pltpu.VMEM((4, R, NOUT), BF16),
            pltpu.VMEM((2, R, NOUT), F32),
            pltpu.VMEM((2, R, NOUT), F32),
            pltpu.VMEM((2, R, NOUT), F32),
            pltpu.VMEM((2, R, NOUT), BF16),
            pltpu.VMEM((2, R, NOUT), BF16),
            pltpu.VMEM((2, R, NOUT), BF16),
            pltpu.SemaphoreType.DMA((4,)),
            pltpu.SemaphoreType.DMA((NX,)),
            pltpu.SemaphoreType.DMA((NY,)),
            pltpu.SemaphoreType.DMA((NY,)),
            pltpu.SemaphoreType.DMA((NZ,)),
            pltpu.SemaphoreType.DMA((NZ,)),
            pltpu.SemaphoreType.DMA((2,)),
            pltpu.SemaphoreType.DMA((2,)),
            pltpu.SemaphoreType.DMA((2,)),
            pltpu.SemaphoreType.DMA((2,)),
            pltpu.SemaphoreType.DMA((2,)),
            pltpu.SemaphoreType.DMA((2,)),
            pltpu.SemaphoreType.DMA((2,)),
        ],
        compiler_params=pltpu.CompilerParams(
            collective_id=0, vmem_limit_bytes=48 * 1024 * 1024),
    )(x)
